# baseline (speedup 1.0000x reference)
import jax
import jax.numpy as jnp
from jax.experimental import pallas as pl
from jax.experimental.pallas import tpu as pltpu

SCALE = 28.0
ALPHA = 0.6
BETA = 0.4
SCALE_POS = 10.0
SCALE_NEG = 40.0
NUM_CLASSES = 11003
FEATURE_SIZE = 2048
BATCH = 1024

_CB = 512
_MC = 512
_NJ = 22
_E8 = 16.0
_RB = 256
_NORM = (2 * BATCH) // _RB
_CC = SCALE / (_E8 * _E8)
_LOG2E = 1.4426950408889634
_C2 = _CC * _LOG2E
_S2 = SCALE * _LOG2E


def _main_body(v_ref, t_ref, w_ref, labb_ref, se_ref, la_ref, en8_ref):
    j = pl.program_id(0)

    @pl.when(j < _NORM)
    def _():
        x = jnp.where(j < _NORM // 2, v_ref[...], t_ref[...])
        n = x * jax.lax.rsqrt(jnp.sum(x * x, axis=1, keepdims=True))
        row0 = pl.multiple_of(j * _RB, _RB)
        en8_ref[pl.ds(row0, _RB), :] = (n * _E8).astype(jnp.float8_e4m3fn)

    @pl.when(j == _NORM)
    def _():
        se_ref[...] = jnp.zeros(se_ref.shape, se_ref.dtype)
        la_ref[...] = jnp.zeros(la_ref.shape, la_ref.dtype)

    @pl.when(j >= _NORM)
    def _():
        w = w_ref[...]
        ssq = jnp.sum(w * w, axis=0, keepdims=True)
        rinv = _E8 * jax.lax.rsqrt(jnp.maximum(ssq, 1e-30))
        w8 = (w * rinv).astype(jnp.float8_e4m3fn)

        col0 = (j - _NORM) * _CB
        cid1 = col0 + jax.lax.broadcasted_iota(jnp.int32, (1, _CB), 1)
        ok = cid1 < NUM_CLASSES

        cid = col0 + jax.lax.broadcasted_iota(jnp.int32, (_MC, _CB), 1)
        for r in range(0, 2 * BATCH, _MC):
            raw = jnp.dot(en8_ref[r:r + _MC, :], w8,
                          preferred_element_type=jnp.float32)
            ex = jnp.exp2(jnp.where(ok, raw * _C2 - _S2, -1e4))
            labm = pltpu.repeat(labb_ref[r:r + _MC, :],
                                _CB // 128, axis=1) == cid
            lv = jnp.where(labm, raw, 0.0)
            exs = [ex[:, k * 128:(k + 1) * 128] for k in range(_CB // 128)]
            lvs = [lv[:, k * 128:(k + 1) * 128] for k in range(_CB // 128)]
            while len(exs) > 1:
                exs = [a + b for a, b in zip(exs[::2], exs[1::2])]
                lvs = [a + b for a, b in zip(lvs[::2], lvs[1::2])]
            se_ref[0, r:r + _MC, :] += exs[0]
            la_ref[0, r:r + _MC, :] += lvs[0]


def _sim_body(t8_ref, v8_ref, labv_ref, labr_ref, se_ref, la_ref,
              o_ref, ga_ref):
    j = pl.program_id(0)

    @pl.when(j == 0)
    def _():
        ga_ref[...] = jnp.zeros(ga_ref.shape, ga_ref.dtype)

    raws = jax.lax.dot_general(t8_ref[...], v8_ref[...],
                               (((1,), (1,)), ((), ())),
                               preferred_element_type=jnp.float32)
    posm = pltpu.repeat(labv_ref[...], BATCH // 128, axis=1) == \
        jnp.broadcast_to(labr_ref[...], raws.shape)
    coef = jnp.where(posm, -SCALE_POS / (_E8 * _E8), SCALE_NEG / (_E8 * _E8))
    off = jnp.where(posm, SCALE_POS * ALPHA, -SCALE_NEG * BETA)
    x = coef * raws + off
    pp = jnp.maximum(x, 0.0) + jnp.log1p(jnp.exp(-jnp.abs(x)))
    prow = jnp.sum(pp, axis=0, keepdims=True)
    acc = ga_ref[...]
    for k in range(BATCH // 128):
        acc = acc + prow[:, k * 128:(k + 1) * 128]
    ga_ref[...] = acc

    @pl.when(j == 1)
    def _():
        s = se_ref[0]
        srow = jnp.sum(s, axis=1, keepdims=True)
        suml = jnp.sum(jnp.log(srow))
        labt = jnp.sum(la_ref[0])
        inst = (suml - _CC * labt) / BATCH + 2.0 * SCALE
        ga = 2.0 * jnp.sum(ga_ref[...]) / BATCH
        o_ref[0] = inst
        o_ref[1] = ga


def kernel(visual_embed, textual_embed, labels, W):
    labels = labels.astype(jnp.int32)
    lab2 = jnp.concatenate([labels, labels], axis=0)
    labb = jnp.broadcast_to(lab2[:, None], (2 * BATCH, 128))
    labv = jnp.broadcast_to(labels[:, None], (BATCH, 128))
    labr = labels[None, :]

    se, la, En8 = pl.pallas_call(
        _main_body,
        grid=(_NJ + _NORM,),
        in_specs=[
            pl.BlockSpec((_RB, FEATURE_SIZE),
                         lambda j: (jnp.minimum(j, _NORM // 2 - 1), 0)),
            pl.BlockSpec((_RB, FEATURE_SIZE),
                         lambda j: (jnp.clip(j - _NORM // 2, 0,
                                             _NORM // 2 - 1), 0)),
            pl.BlockSpec((FEATURE_SIZE, _CB),
                         lambda j: (0, jnp.maximum(j - _NORM, 0))),
            pl.BlockSpec((2 * BATCH, 128), lambda j: (0, 0)),
        ],
        out_specs=[
            pl.BlockSpec((1, 2 * BATCH, 128), lambda j: (0, 0, 0)),
            pl.BlockSpec((1, 2 * BATCH, 128), lambda j: (0, 0, 0)),
            pl.BlockSpec((2 * BATCH, FEATURE_SIZE), lambda j: (0, 0)),
        ],
        out_shape=[
            jax.ShapeDtypeStruct((1, 2 * BATCH, 128), jnp.float32),
            jax.ShapeDtypeStruct((1, 2 * BATCH, 128), jnp.float32),
            jax.ShapeDtypeStruct((2 * BATCH, FEATURE_SIZE),
                                 jnp.float8_e4m3fn),
        ],
        compiler_params=pltpu.CompilerParams(
            dimension_semantics=("arbitrary",),
            vmem_limit_bytes=100 * 1024 * 1024),
        name="cosine_ce",
    )(visual_embed, textual_embed, W, labb)

    out = pl.pallas_call(
        _sim_body,
        grid=(2,),
        in_specs=[
            pl.BlockSpec((BATCH // 2, FEATURE_SIZE), lambda j: (j + 2, 0)),
            pl.BlockSpec((BATCH, FEATURE_SIZE), lambda j: (0, 0)),
            pl.BlockSpec((BATCH // 2, 128), lambda j: (j, 0)),
            pl.BlockSpec((1, BATCH), lambda j: (0, 0)),
            pl.BlockSpec((1, 2 * BATCH, 128), lambda j: (0, 0, 0)),
            pl.BlockSpec((1, 2 * BATCH, 128), lambda j: (0, 0, 0)),
        ],
        out_specs=pl.BlockSpec(memory_space=pltpu.SMEM),
        out_shape=jax.ShapeDtypeStruct((2,), jnp.float32),
        scratch_shapes=[pltpu.VMEM((1, 128), jnp.float32)],
        compiler_params=pltpu.CompilerParams(
            dimension_semantics=("arbitrary",),
            vmem_limit_bytes=100 * 1024 * 1024),
        name="global_align",
    )(En8, En8, labv, labr, se, la)
    return out


# speedup vs baseline: 1.1252x; 1.1252x over previous
import jax
import jax.numpy as jnp
from jax.experimental import pallas as pl
from jax.experimental.pallas import tpu as pltpu

SCALE = 28.0
ALPHA = 0.6
BETA = 0.4
SCALE_POS = 10.0
SCALE_NEG = 40.0
NUM_CLASSES = 11003
FEATURE_SIZE = 2048
BATCH = 1024

_CB = 1024
_MC = 256
_NJ = 11
_E8 = 16.0
_RB = 256
_NORM = (2 * BATCH) // _RB
_CC = SCALE / (_E8 * _E8)
_LOG2E = 1.4426950408889634
_C2 = _CC * _LOG2E
_S2 = SCALE * _LOG2E


def _main_body(v_ref, t_ref, w_ref, labb_ref, se_ref, la_ref, en8_ref):
    j = pl.program_id(0)

    @pl.when(j < _NORM)
    def _():
        x = jnp.where(j < _NORM // 2, v_ref[...], t_ref[...])
        n = x * jax.lax.rsqrt(jnp.sum(x * x, axis=1, keepdims=True))
        row0 = pl.multiple_of(j * _RB, _RB)
        en8_ref[pl.ds(row0, _RB), :] = (n * _E8).astype(jnp.float8_e4m3fn)

    @pl.when(j == _NORM)
    def _():
        se_ref[...] = jnp.zeros(se_ref.shape, se_ref.dtype)
        la_ref[...] = jnp.zeros(la_ref.shape, la_ref.dtype)

    def _block_loop(masked):
        w = w_ref[...]
        ssq = jnp.sum(w * w, axis=0, keepdims=True)
        rinv = _E8 * jax.lax.rsqrt(jnp.maximum(ssq, 1e-30))
        w8 = (w * rinv).astype(jnp.float8_e4m3fn)

        col0 = (j - _NORM) * _CB
        cid1 = col0 + jax.lax.broadcasted_iota(jnp.int32, (1, _CB), 1)
        ok = cid1 < NUM_CLASSES

        cid = col0 + jax.lax.broadcasted_iota(jnp.int32, (_MC, _CB), 1)
        for r in range(0, 2 * BATCH, _MC):
            raw = jnp.dot(en8_ref[r:r + _MC, :], w8,
                          preferred_element_type=jnp.float32)
            sh = raw * _C2 - _S2
            ex = jnp.exp2(jnp.where(ok, sh, -1e4) if masked else sh)
            labm = pltpu.repeat(labb_ref[r:r + _MC, :],
                                _CB // 128, axis=1) == cid
            lv = jnp.where(labm, raw, 0.0)
            exs = [ex[:, k * 128:(k + 1) * 128] for k in range(_CB // 128)]
            lvs = [lv[:, k * 128:(k + 1) * 128] for k in range(_CB // 128)]
            while len(exs) > 1:
                exs = [a + b for a, b in zip(exs[::2], exs[1::2])]
                lvs = [a + b for a, b in zip(lvs[::2], lvs[1::2])]
            se_ref[0, r:r + _MC, :] += exs[0]
            la_ref[0, r:r + _MC, :] += lvs[0]

    @pl.when((j >= _NORM) & (j < _NORM + _NJ - 1))
    def _():
        _block_loop(masked=False)

    @pl.when(j == _NORM + _NJ - 1)
    def _():
        _block_loop(masked=True)


def _sim_body(t8_ref, v8_ref, labv_ref, labr_ref, se_ref, la_ref,
              o_ref, ga_ref):
    j = pl.program_id(0)

    @pl.when(j == 0)
    def _():
        ga_ref[...] = jnp.zeros(ga_ref.shape, ga_ref.dtype)

    raws = jax.lax.dot_general(t8_ref[...], v8_ref[...],
                               (((1,), (1,)), ((), ())),
                               preferred_element_type=jnp.float32)
    posm = pltpu.repeat(labv_ref[...], BATCH // 128, axis=1) == \
        jnp.broadcast_to(labr_ref[...], raws.shape)
    coef = jnp.where(posm, -SCALE_POS / (_E8 * _E8), SCALE_NEG / (_E8 * _E8))
    off = jnp.where(posm, SCALE_POS * ALPHA, -SCALE_NEG * BETA)
    x = coef * raws + off
    pp = jnp.maximum(x, 0.0) + jnp.log1p(jnp.exp(-jnp.abs(x)))
    prow = jnp.sum(pp, axis=0, keepdims=True)
    acc = ga_ref[...]
    for k in range(BATCH // 128):
        acc = acc + prow[:, k * 128:(k + 1) * 128]
    ga_ref[...] = acc

    @pl.when(j == 1)
    def _():
        s = se_ref[0]
        srow = jnp.sum(s, axis=1, keepdims=True)
        suml = jnp.sum(jnp.log(srow))
        labt = jnp.sum(la_ref[0])
        inst = (suml - _CC * labt) / BATCH + 2.0 * SCALE
        ga = 2.0 * jnp.sum(ga_ref[...]) / BATCH
        o_ref[0] = inst
        o_ref[1] = ga


def kernel(visual_embed, textual_embed, labels, W):
    labels = labels.astype(jnp.int32)
    lab2 = jnp.concatenate([labels, labels], axis=0)
    labb = jnp.broadcast_to(lab2[:, None], (2 * BATCH, 128))
    labv = jnp.broadcast_to(labels[:, None], (BATCH, 128))
    labr = labels[None, :]

    se, la, En8 = pl.pallas_call(
        _main_body,
        grid=(_NJ + _NORM,),
        in_specs=[
            pl.BlockSpec((_RB, FEATURE_SIZE),
                         lambda j: (jnp.minimum(j, _NORM // 2 - 1), 0)),
            pl.BlockSpec((_RB, FEATURE_SIZE),
                         lambda j: (jnp.clip(j - _NORM // 2, 0,
                                             _NORM // 2 - 1), 0)),
            pl.BlockSpec((FEATURE_SIZE, _CB),
                         lambda j: (0, jnp.maximum(j - _NORM, 0))),
            pl.BlockSpec((2 * BATCH, 128), lambda j: (0, 0)),
        ],
        out_specs=[
            pl.BlockSpec((1, 2 * BATCH, 128), lambda j: (0, 0, 0)),
            pl.BlockSpec((1, 2 * BATCH, 128), lambda j: (0, 0, 0)),
            pl.BlockSpec((2 * BATCH, FEATURE_SIZE), lambda j: (0, 0)),
        ],
        out_shape=[
            jax.ShapeDtypeStruct((1, 2 * BATCH, 128), jnp.float32),
            jax.ShapeDtypeStruct((1, 2 * BATCH, 128), jnp.float32),
            jax.ShapeDtypeStruct((2 * BATCH, FEATURE_SIZE),
                                 jnp.float8_e4m3fn),
        ],
        compiler_params=pltpu.CompilerParams(
            dimension_semantics=("arbitrary",),
            vmem_limit_bytes=100 * 1024 * 1024),
        name="cosine_ce",
    )(visual_embed, textual_embed, W, labb)

    out = pl.pallas_call(
        _sim_body,
        grid=(2,),
        in_specs=[
            pl.BlockSpec((BATCH // 2, FEATURE_SIZE), lambda j: (j + 2, 0)),
            pl.BlockSpec((BATCH, FEATURE_SIZE), lambda j: (0, 0)),
            pl.BlockSpec((BATCH // 2, 128), lambda j: (j, 0)),
            pl.BlockSpec((1, BATCH), lambda j: (0, 0)),
            pl.BlockSpec((1, 2 * BATCH, 128), lambda j: (0, 0, 0)),
            pl.BlockSpec((1, 2 * BATCH, 128), lambda j: (0, 0, 0)),
        ],
        out_specs=pl.BlockSpec(memory_space=pltpu.SMEM),
        out_shape=jax.ShapeDtypeStruct((2,), jnp.float32),
        scratch_shapes=[pltpu.VMEM((1, 128), jnp.float32)],
        compiler_params=pltpu.CompilerParams(
            dimension_semantics=("arbitrary",),
            vmem_limit_bytes=100 * 1024 * 1024),
        name="global_align",
    )(En8, En8, labv, labr, se, la)
    return out


# speedup vs baseline: 1.1876x; 1.0554x over previous
import jax
import jax.numpy as jnp
from jax.experimental import pallas as pl
from jax.experimental.pallas import tpu as pltpu

SCALE = 28.0
ALPHA = 0.6
BETA = 0.4
SCALE_POS = 10.0
SCALE_NEG = 40.0
NUM_CLASSES = 11003
FEATURE_SIZE = 2048
BATCH = 1024

_CB = 1024
_MC = 256
_NJ = 11
_E8 = 16.0
_RB = 256
_NORM = (2 * BATCH) // _RB
_CC = SCALE / (_E8 * _E8)
_LOG2E = 1.4426950408889634
_C2 = _CC * _LOG2E
_S2 = SCALE * _LOG2E


def _main_body(v_ref, t_ref, w_ref, labb_ref, labv_ref, labr_ref,
               o_ref, se_ref, la_ref, en8_ref, ga_ref):
    j = pl.program_id(0)

    @pl.when(j < _NORM)
    def _():
        x = jnp.where(j < _NORM // 2, v_ref[...], t_ref[...])
        n = x * jax.lax.rsqrt(jnp.sum(x * x, axis=1, keepdims=True))
        row0 = pl.multiple_of(j * _RB, _RB)
        en8_ref[pl.ds(row0, _RB), :] = (n * _E8).astype(jnp.float8_e4m3fn)

    @pl.when(j == _NORM)
    def _():
        se_ref[...] = jnp.zeros(se_ref.shape, se_ref.dtype)
        la_ref[...] = jnp.zeros(la_ref.shape, la_ref.dtype)

    def _block_loop(masked):
        w = w_ref[...]
        ssq = jnp.sum(w * w, axis=0, keepdims=True)
        rinv = _E8 * jax.lax.rsqrt(jnp.maximum(ssq, 1e-30))
        w8 = (w * rinv).astype(jnp.float8_e4m3fn)

        col0 = (j - _NORM) * _CB
        cid1 = col0 + jax.lax.broadcasted_iota(jnp.int32, (1, _CB), 1)
        ok = cid1 < NUM_CLASSES

        cid = col0 + jax.lax.broadcasted_iota(jnp.int32, (_MC, _CB), 1)
        for r in range(0, 2 * BATCH, _MC):
            raw = jnp.dot(en8_ref[r:r + _MC, :], w8,
                          preferred_element_type=jnp.float32)
            sh = raw * _C2 - _S2
            ex = jnp.exp2(jnp.where(ok, sh, -1e4) if masked else sh)
            labm = pltpu.repeat(labb_ref[r:r + _MC, :],
                                _CB // 128, axis=1) == cid
            lv = jnp.where(labm, raw, 0.0)
            exs = [ex[:, k * 128:(k + 1) * 128] for k in range(_CB // 128)]
            lvs = [lv[:, k * 128:(k + 1) * 128] for k in range(_CB // 128)]
            while len(exs) > 1:
                exs = [a + b for a, b in zip(exs[::2], exs[1::2])]
                lvs = [a + b for a, b in zip(lvs[::2], lvs[1::2])]
            se_ref[r:r + _MC, :] += exs[0]
            la_ref[r:r + _MC, :] += lvs[0]

    @pl.when((j >= _NORM) & (j < _NORM + _NJ - 1))
    def _():
        _block_loop(masked=False)

    @pl.when(j == _NORM + _NJ - 1)
    def _():
        _block_loop(masked=True)

    def _sim_half(h):
        t8 = en8_ref[BATCH + h * (BATCH // 2):
                     BATCH + (h + 1) * (BATCH // 2), :]
        raws = jax.lax.dot_general(t8, en8_ref[0:BATCH, :],
                                   (((1,), (1,)), ((), ())),
                                   preferred_element_type=jnp.float32)
        labv = labv_ref[h * (BATCH // 2):(h + 1) * (BATCH // 2), :]
        posm = pltpu.repeat(labv, BATCH // 128, axis=1) == \
            jnp.broadcast_to(labr_ref[...], raws.shape)
        coef = jnp.where(posm, -SCALE_POS / (_E8 * _E8),
                         SCALE_NEG / (_E8 * _E8))
        off = jnp.where(posm, SCALE_POS * ALPHA, -SCALE_NEG * BETA)
        x = coef * raws + off
        pp = jnp.maximum(x, 0.0) + jnp.log1p(jnp.exp(-jnp.abs(x)))
        prow = jnp.sum(pp, axis=0, keepdims=True)
        acc = ga_ref[...]
        for k in range(BATCH // 128):
            acc = acc + prow[:, k * 128:(k + 1) * 128]
        ga_ref[...] = acc

    @pl.when(j == _NORM)
    def _():
        ga_ref[...] = jnp.zeros(ga_ref.shape, ga_ref.dtype)
        _sim_half(0)

    @pl.when(j == _NORM + 1)
    def _():
        _sim_half(1)

    @pl.when(j == _NORM + _NJ - 1)
    def _():
        srow = jnp.sum(se_ref[...], axis=1, keepdims=True)
        suml = jnp.sum(jnp.log(srow))
        labt = jnp.sum(la_ref[...])
        inst = (suml - _CC * labt) / BATCH + 2.0 * SCALE
        ga = 2.0 * jnp.sum(ga_ref[...]) / BATCH
        o_ref[0] = inst
        o_ref[1] = ga


def _sim_body(t8_ref, v8_ref, labv_ref, labr_ref, se_ref, la_ref,
              o_ref, ga_ref):
    j = pl.program_id(0)

    @pl.when(j == 0)
    def _():
        ga_ref[...] = jnp.zeros(ga_ref.shape, ga_ref.dtype)

    raws = jax.lax.dot_general(t8_ref[...], v8_ref[...],
                               (((1,), (1,)), ((), ())),
                               preferred_element_type=jnp.float32)
    posm = pltpu.repeat(labv_ref[...], BATCH // 128, axis=1) == \
        jnp.broadcast_to(labr_ref[...], raws.shape)
    coef = jnp.where(posm, -SCALE_POS / (_E8 * _E8), SCALE_NEG / (_E8 * _E8))
    off = jnp.where(posm, SCALE_POS * ALPHA, -SCALE_NEG * BETA)
    x = coef * raws + off
    pp = jnp.maximum(x, 0.0) + jnp.log1p(jnp.exp(-jnp.abs(x)))
    prow = jnp.sum(pp, axis=0, keepdims=True)
    acc = ga_ref[...]
    for k in range(BATCH // 128):
        acc = acc + prow[:, k * 128:(k + 1) * 128]
    ga_ref[...] = acc

    @pl.when(j == 1)
    def _():
        s = se_ref[0]
        srow = jnp.sum(s, axis=1, keepdims=True)
        suml = jnp.sum(jnp.log(srow))
        labt = jnp.sum(la_ref[0])
        inst = (suml - _CC * labt) / BATCH + 2.0 * SCALE
        ga = 2.0 * jnp.sum(ga_ref[...]) / BATCH
        o_ref[0] = inst
        o_ref[1] = ga


def kernel(visual_embed, textual_embed, labels, W):
    labels = labels.astype(jnp.int32)
    lab2 = jnp.concatenate([labels, labels], axis=0)
    labb = jnp.broadcast_to(lab2[:, None], (2 * BATCH, 128))
    labv = jnp.broadcast_to(labels[:, None], (BATCH, 128))
    labr = labels[None, :]

    out = pl.pallas_call(
        _main_body,
        grid=(_NJ + _NORM,),
        in_specs=[
            pl.BlockSpec((_RB, FEATURE_SIZE),
                         lambda j: (jnp.minimum(j, _NORM // 2 - 1), 0)),
            pl.BlockSpec((_RB, FEATURE_SIZE),
                         lambda j: (jnp.clip(j - _NORM // 2, 0,
                                             _NORM // 2 - 1), 0)),
            pl.BlockSpec((FEATURE_SIZE, _CB),
                         lambda j: (0, jnp.maximum(j - _NORM, 0))),
            pl.BlockSpec((2 * BATCH, 128), lambda j: (0, 0)),
            pl.BlockSpec((BATCH, 128), lambda j: (0, 0)),
            pl.BlockSpec((1, BATCH), lambda j: (0, 0)),
        ],
        out_specs=pl.BlockSpec(memory_space=pltpu.SMEM),
        out_shape=jax.ShapeDtypeStruct((2,), jnp.float32),
        scratch_shapes=[
            pltpu.VMEM((2 * BATCH, 128), jnp.float32),
            pltpu.VMEM((2 * BATCH, 128), jnp.float32),
            pltpu.VMEM((2 * BATCH, FEATURE_SIZE), jnp.float8_e4m3fn),
            pltpu.VMEM((1, 128), jnp.float32),
        ],
        compiler_params=pltpu.CompilerParams(
            dimension_semantics=("arbitrary",),
            vmem_limit_bytes=100 * 1024 * 1024),
        name="cosine_ce",
    )(visual_embed, textual_embed, W, labb, labv, labr)
    return out
